# baseline (speedup 1.0000x reference)
import functools

import jax
import jax.numpy as jnp
from jax import lax
from jax.experimental import pallas as pl
from jax.experimental.pallas import tpu as pltpu
from jax.experimental.pallas import tpu_sc as plsc

NEG_SLOPE = 0.2
EPS = 1e-5
GROUPS = 4
NW = 32


def _round_up(x, m):
    return (x + m - 1) // m * m


def _sc_gather(table, idx):
    T, C = table.shape
    (M,) = idx.shape
    r = M // NW
    ch = min(2048, (57000 // (C + 1)) // 8 * 8)
    ch = min(ch, r)
    nch = -(-r // ch)
    starts = [min(c * ch, r - ch) for c in range(nch)]
    mesh = plsc.VectorSubcoreMesh(core_axis_name="c", subcore_axis_name="s")

    @functools.partial(
        pl.kernel,
        mesh=mesh,
        out_type=jax.ShapeDtypeStruct((M, C), jnp.float32),
        compiler_params=pltpu.CompilerParams(use_tc_tiling_on_sc=False),
        scratch_types=[
            pltpu.VMEM((2, ch), jnp.int32),
            pltpu.VMEM((2, ch, C), jnp.float32),
            pltpu.SemaphoreType.DMA((2,)),
            pltpu.SemaphoreType.DMA((2,)),
            pltpu.SemaphoreType.DMA((2,)),
        ],
    )
    def gk(table_hbm, idx_hbm, out_hbm, idx_v, rows_v, sem_i, sem_g, sem_o):
        wid = lax.axis_index("s") * 2 + lax.axis_index("c")
        base = wid * r

        def idx_cp(c):
            b = c % 2
            return pltpu.make_async_copy(
                idx_hbm.at[pl.ds(base + starts[c], ch)], idx_v.at[b], sem_i.at[b])

        def gat_cp(c):
            b = c % 2
            return pltpu.make_async_copy(
                table_hbm.at[idx_v.at[b]], rows_v.at[b], sem_g.at[b])

        def out_cp(c):
            b = c % 2
            return pltpu.make_async_copy(
                rows_v.at[b], out_hbm.at[pl.ds(base + starts[c], ch)], sem_o.at[b])

        idx_cp(0).start()
        idx_cp(0).wait()
        gat_cp(0).start()
        for c in range(nch):
            gat_cp(c).wait()
            if c + 1 < nch:
                idx_cp(c + 1).start()
                if c >= 1:
                    out_cp(c - 1).wait()
                idx_cp(c + 1).wait()
                gat_cp(c + 1).start()
            out_cp(c).start()
        if nch >= 2:
            out_cp(nch - 2).wait()
        out_cp(nch - 1).wait()

    return gk(table, idx)


def _sc_pool_conv(F2, pidx, cidx):
    Tprev, C = F2.shape
    (Mp,) = cidx.shape
    n_pad = Mp // 7
    u = min(max(32, 4096 // C), n_pad // 16)
    R = n_pad // 16
    nch1 = -(-R // u)
    st1 = [min(c * u, R - u) for c in range(nch1)]
    r2 = Mp // NW
    ch = min(2048, (41000 // (C + 1)) // 8 * 8)
    ch = min(ch, r2)
    nch2 = -(-r2 // ch)
    st2 = [min(c * ch, r2 - ch) for c in range(nch2)]
    mesh = plsc.VectorSubcoreMesh(core_axis_name="c", subcore_axis_name="s")

    @functools.partial(
        pl.kernel,
        mesh=mesh,
        out_type=[
            jax.ShapeDtypeStruct((Mp, C), jnp.float32),
            jax.ShapeDtypeStruct((2, n_pad, C), jnp.float32),
        ],
        compiler_params=pltpu.CompilerParams(use_tc_tiling_on_sc=False),
        scratch_types=[
            pltpu.VMEM((7 * u,), jnp.int32),
            pltpu.VMEM((7 * u, C), jnp.float32),
            pltpu.VMEM((u, C), jnp.float32),
            pltpu.VMEM((2, ch), jnp.int32),
            pltpu.VMEM((2, ch, C), jnp.float32),
            pltpu.SemaphoreType.DMA,
            pltpu.SemaphoreType.DMA((2,)),
            pltpu.SemaphoreType.DMA((2,)),
            pltpu.SemaphoreType.DMA((2,)),
        ],
    )
    def k(f2_hbm, pidx_hbm, cidx_hbm, g_hbm, pool_hbm,
          pi_v, prow_v, pbuf_v, idx_v, rows_v, sem1, sem_i, sem_g, sem_o):
        ci = lax.axis_index("c")
        sid = lax.axis_index("s")
        tbase = sid * R
        for s0 in st1:
            row0 = tbase + s0
            pltpu.sync_copy(pidx_hbm.at[pl.ds(row0 * 7, 7 * u)], pi_v)
            pltpu.async_copy(f2_hbm.at[pi_v], prow_v, sem1).wait()

            def body(j, _):
                for cc in range(C // 16):
                    sl = pl.ds(cc * 16, 16)
                    acc = prow_v[7 * j, sl]
                    for kk in range(1, 7):
                        acc = acc + prow_v[7 * j + kk, sl]
                    pbuf_v[j, sl] = acc * (1.0 / 7.0)
                return 0

            lax.fori_loop(0, u, body, 0)
            pltpu.sync_copy(pbuf_v, pool_hbm.at[ci, pl.ds(row0, u)])
        plsc.subcore_barrier()
        wid = sid * 2 + ci
        base = wid * r2
        table = pool_hbm.at[ci]

        def idx_cp(c):
            b = c % 2
            return pltpu.make_async_copy(
                cidx_hbm.at[pl.ds(base + st2[c], ch)], idx_v.at[b], sem_i.at[b])

        def gat_cp(c):
            b = c % 2
            return pltpu.make_async_copy(
                table.at[idx_v.at[b]], rows_v.at[b], sem_g.at[b])

        def out_cp(c):
            b = c % 2
            return pltpu.make_async_copy(
                rows_v.at[b], g_hbm.at[pl.ds(base + st2[c], ch)], sem_o.at[b])

        idx_cp(0).start()
        idx_cp(0).wait()
        gat_cp(0).start()
        for c in range(nch2):
            gat_cp(c).wait()
            if c + 1 < nch2:
                idx_cp(c + 1).start()
                if c >= 1:
                    out_cp(c - 1).wait()
                idx_cp(c + 1).wait()
                gat_cp(c + 1).start()
            out_cp(c).start()
        if nch2 >= 2:
            out_cp(nch2 - 2).wait()
        out_cp(nch2 - 1).wait()

    return k(F2, pidx, cidx)


def _affine_from_stats(stats, gamma, beta, n_prev, C):
    gc = C // GROUPS
    cidx = lax.broadcasted_iota(jnp.int32, (C, GROUPS), 0)
    gidx = lax.broadcasted_iota(jnp.int32, (C, GROUPS), 1)
    ind = (cidx // gc == gidx).astype(jnp.float32)
    gsum = jnp.dot(stats, ind, preferred_element_type=jnp.float32)
    cnt = float(gc * n_prev)
    mean_g = gsum[0:1] / cnt
    var_g = gsum[1:2] / cnt - mean_g * mean_g
    mean_c = jnp.dot(mean_g, ind.T, preferred_element_type=jnp.float32)
    var_c = jnp.dot(var_g, ind.T, preferred_element_type=jnp.float32)
    v = var_c + EPS
    r = lax.rsqrt(v)
    r = r * (1.5 - 0.5 * v * r * r)
    s = gamma * r
    t = beta - mean_c * s
    return s, t


def _pick_bn(n_pad):
    return 1536 if n_pad % 1536 == 0 else n_pad


def _tc_conv(G2, W, b, n_real, n_prev, stats, gamma, beta):
    n_pad, K = G2.shape
    O = W.shape[1]
    C = K // 7
    has_aff = stats is not None
    bn = _pick_bn(n_pad)
    S = n_pad // bn

    def body(*refs):
        if has_aff:
            g_ref, w_ref, b_ref, st_ref, ga_ref, be_ref, y_ref, so_ref = refs
        else:
            g_ref, w_ref, b_ref, y_ref, so_ref = refs
        step = pl.program_id(0)
        A = g_ref[...]
        if has_aff:
            s, t = _affine_from_stats(st_ref[...], ga_ref[...], be_ref[...], n_prev, C)
            s7 = jnp.concatenate([s] * 7, axis=1)
            t7 = jnp.concatenate([t] * 7, axis=1)
            A = A * s7 + t7
            A = jnp.where(A >= 0, A, NEG_SLOPE * A)
        Y = jnp.dot(A, w_ref[...], preferred_element_type=jnp.float32,
                    precision=lax.Precision.HIGHEST) + b_ref[...]
        y_ref[...] = Y
        rows = lax.broadcasted_iota(jnp.int32, (bn, 1), 0) + step * bn
        Ym = jnp.where(rows < n_real, Y, 0.0)

        @pl.when(step == 0)
        def _():
            so_ref[...] = jnp.zeros_like(so_ref)

        so_ref[0:1, :] += jnp.sum(Ym, axis=0, keepdims=True)
        so_ref[1:2, :] += jnp.sum(Ym * Ym, axis=0, keepdims=True)

    in_specs = [
        pl.BlockSpec((bn, K), lambda s: (s, 0)),
        pl.BlockSpec((K, O), lambda s: (0, 0)),
        pl.BlockSpec((1, O), lambda s: (0, 0)),
    ]
    args = [G2, W, b.reshape(1, O)]
    if has_aff:
        in_specs += [
            pl.BlockSpec((2, C), lambda s: (0, 0)),
            pl.BlockSpec((1, C), lambda s: (0, 0)),
            pl.BlockSpec((1, C), lambda s: (0, 0)),
        ]
        args += [stats, gamma.reshape(1, C), beta.reshape(1, C)]
    return pl.pallas_call(
        body,
        grid=(S,),
        in_specs=in_specs,
        out_specs=[
            pl.BlockSpec((bn, O), lambda s: (s, 0)),
            pl.BlockSpec((2, O), lambda s: (0, 0)),
        ],
        out_shape=[
            jax.ShapeDtypeStruct((n_pad, O), jnp.float32),
            jax.ShapeDtypeStruct((2, O), jnp.float32),
        ],
    )(*args)


def _tc_finalize(Y, n_prev, stats, gamma, beta):
    n_pad, C = Y.shape
    bn = _pick_bn(n_pad)
    S = n_pad // bn

    def body(y_ref, st_ref, ga_ref, be_ref, o_ref):
        s, t = _affine_from_stats(st_ref[...], ga_ref[...], be_ref[...], n_prev, C)
        A = y_ref[...] * s + t
        o_ref[...] = jnp.where(A >= 0, A, NEG_SLOPE * A)

    return pl.pallas_call(
        body,
        grid=(S,),
        in_specs=[
            pl.BlockSpec((bn, C), lambda s: (s, 0)),
            pl.BlockSpec((2, C), lambda s: (0, 0)),
            pl.BlockSpec((1, C), lambda s: (0, 0)),
            pl.BlockSpec((1, C), lambda s: (0, 0)),
        ],
        out_specs=pl.BlockSpec((bn, C), lambda s: (s, 0)),
        out_shape=jax.ShapeDtypeStruct((n_pad, C), jnp.float32),
    )(Y, stats, gamma.reshape(1, C), beta.reshape(1, C))


def _tc_pool(Gp2, n_prev, stats, gamma, beta):
    n_pad, K = Gp2.shape
    C = K // 7
    bn = _pick_bn(n_pad)
    S = n_pad // bn

    def body(g_ref, st_ref, ga_ref, be_ref, o_ref):
        s, t = _affine_from_stats(st_ref[...], ga_ref[...], be_ref[...], n_prev, C)
        s7 = jnp.concatenate([s] * 7, axis=1)
        t7 = jnp.concatenate([t] * 7, axis=1)
        A = g_ref[...] * s7 + t7
        A = jnp.where(A >= 0, A, NEG_SLOPE * A)
        acc = A[:, 0:C]
        for k in range(1, 7):
            acc = acc + A[:, k * C:(k + 1) * C]
        o_ref[...] = acc * (1.0 / 7.0)

    return pl.pallas_call(
        body,
        grid=(S,),
        in_specs=[
            pl.BlockSpec((bn, K), lambda s: (s, 0)),
            pl.BlockSpec((2, C), lambda s: (0, 0)),
            pl.BlockSpec((1, C), lambda s: (0, 0)),
            pl.BlockSpec((1, C), lambda s: (0, 0)),
        ],
        out_specs=pl.BlockSpec((bn, C), lambda s: (s, 0)),
        out_shape=jax.ShapeDtypeStruct((n_pad, C), jnp.float32),
    )(Gp2, stats, gamma.reshape(1, C), beta.reshape(1, C))


def _tc_head(Y2, n_real, stats, gamma, beta, Wout, bout):
    n_pad, C = Y2.shape
    O = Wout.shape[1]

    def body(y_ref, st_ref, ga_ref, be_ref, wo_ref, bo_ref, o_ref):
        s, t = _affine_from_stats(st_ref[...], ga_ref[...], be_ref[...], n_real, C)
        A = y_ref[...] * s + t
        A = jnp.where(A >= 0, A, NEG_SLOPE * A)
        rows = lax.broadcasted_iota(jnp.int32, (n_pad, 1), 0)
        A = jnp.where(rows < n_real, A, 0.0)
        m = jnp.sum(A, axis=0, keepdims=True) * (1.0 / n_real)
        o_ref[...] = jnp.dot(m, wo_ref[...], preferred_element_type=jnp.float32,
                             precision=lax.Precision.HIGHEST) + bo_ref[...]

    return pl.pallas_call(
        body,
        in_specs=[
            pl.BlockSpec((n_pad, C), lambda: (0, 0)),
            pl.BlockSpec((2, C), lambda: (0, 0)),
            pl.BlockSpec((1, C), lambda: (0, 0)),
            pl.BlockSpec((1, C), lambda: (0, 0)),
            pl.BlockSpec((C, O), lambda: (0, 0)),
            pl.BlockSpec((1, O), lambda: (0, 0)),
        ],
        out_specs=pl.BlockSpec((1, O), lambda: (0, 0)),
        out_shape=jax.ShapeDtypeStruct((1, O), jnp.float32),
    )(Y2, stats, gamma.reshape(1, C), beta.reshape(1, C), Wout, bout.reshape(1, O))


def kernel(x, params, neigh_orders):
    Ns = [no.shape[0] // 7 for no in neigh_orders]
    n_pads = [_round_up(n, 1536) if n >= 1536 else _round_up(n, 1024) for n in Ns]
    n_levels = len(Ns)

    def pad_idx(idx, n_pad):
        m = 7 * n_pad
        return jnp.pad(idx, (0, m - idx.shape[0]))

    c0 = x.shape[1]
    c0p = 16
    feat = jnp.pad(x[0].T, ((0, n_pads[0] - Ns[0]), (0, c0p - c0)))
    feat_is_raw = False
    feat_stats = None
    feat_gamma = feat_beta = None
    feat_nreal = Ns[0]

    for i in range(n_levels):
        blk = params["blocks"][i]
        n, n_pad = Ns[i], n_pads[i]
        idx = pad_idx(neigh_orders[i], n_pad)
        if i > 0:
            F2 = _tc_finalize(feat, feat_nreal, feat_stats, feat_gamma, feat_beta)
            idxp = pad_idx(neigh_orders[i - 1][: 7 * n], n_pad)
            G, _pooled = _sc_pool_conv(F2, idxp, idx)
            C = F2.shape[1]
            G2 = G.reshape(n_pad, 7 * C)
            Y1, st1 = _tc_conv(G2, blk["W1"], blk["b1"], n, n, None, None, None)
            feat_nreal = n
        else:
            C = feat.shape[1]
            W1 = blk["W1"]
            if c0p != c0:
                W1 = jnp.pad(W1.reshape(7, c0, -1), ((0, 0), (0, c0p - c0), (0, 0)))
                W1 = W1.reshape(7 * c0p, -1)
            G = _sc_gather(feat, idx)
            G2 = G.reshape(n_pad, 7 * C)
            Y1, st1 = _tc_conv(G2, W1, blk["b1"], n, feat_nreal, None, None, None)

        C1 = Y1.shape[1]
        G = _sc_gather(Y1, idx)
        G2 = G.reshape(n_pad, 7 * C1)
        Y2, st2 = _tc_conv(G2, blk["W2"], blk["b2"], n, n, st1, blk["g1"], blk["be1"])

        feat = Y2
        feat_is_raw = True
        feat_stats, feat_gamma, feat_beta = st2, blk["g2"], blk["be2"]
        feat_nreal = n

    out = _tc_head(
        feat, feat_nreal, feat_stats, feat_gamma, feat_beta,
        params["Wout"], params["bout"],
    )
    return out.reshape(1, 3, 1)


# speedup vs baseline: 1.0989x; 1.0989x over previous
import functools

import jax
import jax.numpy as jnp
from jax import lax
from jax.experimental import pallas as pl
from jax.experimental.pallas import tpu as pltpu
from jax.experimental.pallas import tpu_sc as plsc

NEG_SLOPE = 0.2
EPS = 1e-5
GROUPS = 4
NW = 32


def _round_up(x, m):
    return (x + m - 1) // m * m


def _sc_gather(table, idx):
    T, C = table.shape
    (M,) = idx.shape
    r = M // NW
    ch = min(2048, (57000 // (C + 1)) // 8 * 8)
    ch = min(ch, r)
    nch = -(-r // ch)
    starts = [min(c * ch, r - ch) for c in range(nch)]
    mesh = plsc.VectorSubcoreMesh(core_axis_name="c", subcore_axis_name="s")

    @functools.partial(
        pl.kernel,
        mesh=mesh,
        out_type=jax.ShapeDtypeStruct((M, C), jnp.float32),
        compiler_params=pltpu.CompilerParams(use_tc_tiling_on_sc=False),
        scratch_types=[
            pltpu.VMEM((2, ch), jnp.int32),
            pltpu.VMEM((2, ch, C), jnp.float32),
            pltpu.SemaphoreType.DMA((2,)),
            pltpu.SemaphoreType.DMA((2,)),
            pltpu.SemaphoreType.DMA((2,)),
        ],
    )
    def gk(table_hbm, idx_hbm, out_hbm, idx_v, rows_v, sem_i, sem_g, sem_o):
        wid = lax.axis_index("s") * 2 + lax.axis_index("c")
        base = wid * r

        def idx_cp(c):
            b = c % 2
            return pltpu.make_async_copy(
                idx_hbm.at[pl.ds(base + starts[c], ch)], idx_v.at[b], sem_i.at[b])

        def gat_cp(c):
            b = c % 2
            return pltpu.make_async_copy(
                table_hbm.at[idx_v.at[b]], rows_v.at[b], sem_g.at[b])

        def out_cp(c):
            b = c % 2
            return pltpu.make_async_copy(
                rows_v.at[b], out_hbm.at[pl.ds(base + starts[c], ch)], sem_o.at[b])

        idx_cp(0).start()
        idx_cp(0).wait()
        gat_cp(0).start()
        for c in range(nch):
            gat_cp(c).wait()
            if c + 1 < nch:
                idx_cp(c + 1).start()
                if c >= 1:
                    out_cp(c - 1).wait()
                idx_cp(c + 1).wait()
                gat_cp(c + 1).start()
            out_cp(c).start()
        if nch >= 2:
            out_cp(nch - 2).wait()
        out_cp(nch - 1).wait()

    return gk(table, idx)


def _affine_from_stats(stats, gamma, beta, n_prev, C):
    gc = C // GROUPS
    cidx = lax.broadcasted_iota(jnp.int32, (C, GROUPS), 0)
    gidx = lax.broadcasted_iota(jnp.int32, (C, GROUPS), 1)
    ind = (cidx // gc == gidx).astype(jnp.float32)
    gsum = jnp.dot(stats, ind, preferred_element_type=jnp.float32)
    cnt = float(gc * n_prev)
    mean_g = gsum[0:1] / cnt
    var_g = gsum[1:2] / cnt - mean_g * mean_g
    mean_c = jnp.dot(mean_g, ind.T, preferred_element_type=jnp.float32)
    var_c = jnp.dot(var_g, ind.T, preferred_element_type=jnp.float32)
    v = var_c + EPS
    r = lax.rsqrt(v)
    r = r * (1.5 - 0.5 * v * r * r)
    s = gamma * r
    t = beta - mean_c * s
    return s, t


def _pick_bn(n_pad):
    return 1536 if n_pad % 1536 == 0 else n_pad


def _tc_conv(G2, W, b, n_real, n_prev, stats, gamma, beta):
    n_pad, K = G2.shape
    O = W.shape[1]
    C = K // 7
    has_aff = stats is not None
    bn = _pick_bn(n_pad)
    S = n_pad // bn

    def body(*refs):
        if has_aff:
            g_ref, w_ref, b_ref, st_ref, ga_ref, be_ref, y_ref, so_ref = refs
        else:
            g_ref, w_ref, b_ref, y_ref, so_ref = refs
        step = pl.program_id(0)
        A = g_ref[...]
        if has_aff:
            s, t = _affine_from_stats(st_ref[...], ga_ref[...], be_ref[...], n_prev, C)
            s7 = jnp.concatenate([s] * 7, axis=1)
            t7 = jnp.concatenate([t] * 7, axis=1)
            A = A * s7 + t7
            A = jnp.where(A >= 0, A, NEG_SLOPE * A)
        Y = jnp.dot(A, w_ref[...], preferred_element_type=jnp.float32,
                    precision=lax.Precision.HIGHEST) + b_ref[...]
        y_ref[...] = Y
        rows = lax.broadcasted_iota(jnp.int32, (bn, 1), 0) + step * bn
        Ym = jnp.where(rows < n_real, Y, 0.0)

        @pl.when(step == 0)
        def _():
            so_ref[...] = jnp.zeros_like(so_ref)

        so_ref[0:1, :] += jnp.sum(Ym, axis=0, keepdims=True)
        so_ref[1:2, :] += jnp.sum(Ym * Ym, axis=0, keepdims=True)

    in_specs = [
        pl.BlockSpec((bn, K), lambda s: (s, 0)),
        pl.BlockSpec((K, O), lambda s: (0, 0)),
        pl.BlockSpec((1, O), lambda s: (0, 0)),
    ]
    args = [G2, W, b.reshape(1, O)]
    if has_aff:
        in_specs += [
            pl.BlockSpec((2, C), lambda s: (0, 0)),
            pl.BlockSpec((1, C), lambda s: (0, 0)),
            pl.BlockSpec((1, C), lambda s: (0, 0)),
        ]
        args += [stats, gamma.reshape(1, C), beta.reshape(1, C)]
    return pl.pallas_call(
        body,
        grid=(S,),
        in_specs=in_specs,
        out_specs=[
            pl.BlockSpec((bn, O), lambda s: (s, 0)),
            pl.BlockSpec((2, O), lambda s: (0, 0)),
        ],
        out_shape=[
            jax.ShapeDtypeStruct((n_pad, O), jnp.float32),
            jax.ShapeDtypeStruct((2, O), jnp.float32),
        ],
    )(*args)


def _tc_pool(Gp2, n_prev, stats, gamma, beta):
    n_pad, K = Gp2.shape
    C = K // 7
    bn = _pick_bn(n_pad)
    S = n_pad // bn

    def body(g_ref, st_ref, ga_ref, be_ref, o_ref):
        s, t = _affine_from_stats(st_ref[...], ga_ref[...], be_ref[...], n_prev, C)
        s7 = jnp.concatenate([s] * 7, axis=1)
        t7 = jnp.concatenate([t] * 7, axis=1)
        A = g_ref[...] * s7 + t7
        A = jnp.where(A >= 0, A, NEG_SLOPE * A)
        acc = A[:, 0:C]
        for k in range(1, 7):
            acc = acc + A[:, k * C:(k + 1) * C]
        o_ref[...] = acc * (1.0 / 7.0)

    return pl.pallas_call(
        body,
        grid=(S,),
        in_specs=[
            pl.BlockSpec((bn, K), lambda s: (s, 0)),
            pl.BlockSpec((2, C), lambda s: (0, 0)),
            pl.BlockSpec((1, C), lambda s: (0, 0)),
            pl.BlockSpec((1, C), lambda s: (0, 0)),
        ],
        out_specs=pl.BlockSpec((bn, C), lambda s: (s, 0)),
        out_shape=jax.ShapeDtypeStruct((n_pad, C), jnp.float32),
    )(Gp2, stats, gamma.reshape(1, C), beta.reshape(1, C))


def _tc_head(Y2, n_real, stats, gamma, beta, Wout, bout):
    n_pad, C = Y2.shape
    O = Wout.shape[1]

    def body(y_ref, st_ref, ga_ref, be_ref, wo_ref, bo_ref, o_ref):
        s, t = _affine_from_stats(st_ref[...], ga_ref[...], be_ref[...], n_real, C)
        A = y_ref[...] * s + t
        A = jnp.where(A >= 0, A, NEG_SLOPE * A)
        rows = lax.broadcasted_iota(jnp.int32, (n_pad, 1), 0)
        A = jnp.where(rows < n_real, A, 0.0)
        m = jnp.sum(A, axis=0, keepdims=True) * (1.0 / n_real)
        o_ref[...] = jnp.dot(m, wo_ref[...], preferred_element_type=jnp.float32,
                             precision=lax.Precision.HIGHEST) + bo_ref[...]

    return pl.pallas_call(
        body,
        in_specs=[
            pl.BlockSpec((n_pad, C), lambda: (0, 0)),
            pl.BlockSpec((2, C), lambda: (0, 0)),
            pl.BlockSpec((1, C), lambda: (0, 0)),
            pl.BlockSpec((1, C), lambda: (0, 0)),
            pl.BlockSpec((C, O), lambda: (0, 0)),
            pl.BlockSpec((1, O), lambda: (0, 0)),
        ],
        out_specs=pl.BlockSpec((1, O), lambda: (0, 0)),
        out_shape=jax.ShapeDtypeStruct((1, O), jnp.float32),
    )(Y2, stats, gamma.reshape(1, C), beta.reshape(1, C), Wout, bout.reshape(1, O))


def kernel(x, params, neigh_orders):
    Ns = [no.shape[0] // 7 for no in neigh_orders]
    n_pads = [_round_up(n, 1536) if n >= 1536 else _round_up(n, 1024) for n in Ns]
    n_levels = len(Ns)

    def pad_idx(idx, n_pad):
        m = 7 * n_pad
        return jnp.pad(idx, (0, m - idx.shape[0]))

    c0 = x.shape[1]
    c0p = 16
    feat = jnp.pad(x[0].T, ((0, n_pads[0] - Ns[0]), (0, c0p - c0)))
    feat_is_raw = False
    feat_stats = None
    feat_gamma = feat_beta = None
    feat_nreal = Ns[0]

    for i in range(n_levels):
        blk = params["blocks"][i]
        n, n_pad = Ns[i], n_pads[i]
        if i > 0:
            idxp = pad_idx(neigh_orders[i - 1][: 7 * n], n_pad)
            Gp = _sc_gather(feat, idxp)
            Gp2 = Gp.reshape(n_pad, 7 * feat.shape[1])
            feat = _tc_pool(Gp2, feat_nreal, feat_stats, feat_gamma, feat_beta)
            feat_is_raw = False
            feat_nreal = n

        idx = pad_idx(neigh_orders[i], n_pad)
        C = feat.shape[1]

        W1 = blk["W1"]
        if i == 0 and c0p != c0:
            W1 = jnp.pad(W1.reshape(7, c0, -1), ((0, 0), (0, c0p - c0), (0, 0)))
            W1 = W1.reshape(7 * c0p, -1)
        G = _sc_gather(feat, idx)
        G2 = G.reshape(n_pad, 7 * C)
        Y1, st1 = _tc_conv(
            G2, W1, blk["b1"], n, feat_nreal,
            feat_stats if feat_is_raw else None, feat_gamma, feat_beta,
        )

        C1 = Y1.shape[1]
        G = _sc_gather(Y1, idx)
        G2 = G.reshape(n_pad, 7 * C1)
        Y2, st2 = _tc_conv(G2, blk["W2"], blk["b2"], n, n, st1, blk["g1"], blk["be1"])

        feat = Y2
        feat_is_raw = True
        feat_stats, feat_gamma, feat_beta = st2, blk["g2"], blk["be2"]
        feat_nreal = n

    out = _tc_head(
        feat, feat_nreal, feat_stats, feat_gamma, feat_beta,
        params["Wout"], params["bout"],
    )
    return out.reshape(1, 3, 1)
